# baseline (speedup 1.0000x reference)
import functools

import jax
import jax.numpy as jnp
from jax import lax
from jax.experimental import pallas as pl
from jax.experimental.pallas import tpu as pltpu
from jax.experimental.pallas import tpu_sc as plsc


def kernel(style_ids, frame_ids, latents, latents_mu):
    S, F, D = latents.shape
    B = style_ids.shape[0]
    flat_table = latents.reshape(S * F, D)
    style_ids = style_ids.astype(jnp.int32)
    frame_ids = frame_ids.astype(jnp.int32)

    info = plsc.get_sparse_core_info()
    NC, NS, L = info.num_cores, info.num_subcores, info.num_lanes
    NW = NC * NS
    b_per_w = B // NW
    H = b_per_w // 2

    mesh = plsc.VectorSubcoreMesh(core_axis_name="c", subcore_axis_name="s")

    @functools.partial(
        pl.kernel,
        mesh=mesh,
        out_type=jax.ShapeDtypeStruct((B, D), jnp.float32),
        scratch_types=[
            pltpu.VMEM((b_per_w,), jnp.int32),
            pltpu.VMEM((b_per_w,), jnp.int32),
            pltpu.VMEM((96, D), jnp.float32),
            pltpu.VMEM((160, D), jnp.float32),
            pltpu.VMEM((160, D), jnp.float32),
            pltpu.VMEM((96, D), jnp.float32),
            pltpu.VMEM_SHARED((S, D), jnp.float32),
            pltpu.SemaphoreType.DMA,
            pltpu.SemaphoreType.DMA,
            pltpu.SemaphoreType.DMA,
            pltpu.SemaphoreType.DMA,
            pltpu.SemaphoreType.DMA,
        ],
    )
    def run(style_hbm, frame_hbm, table_hbm, mu_hbm, out_hbm,
            sty_v, idx_v, b0, b1, b2, b3, mu_sh, s0, s1, s2, s3, s_st):
        sid = lax.axis_index("s")
        wid = sid * NC + lax.axis_index("c")
        base = wid * b_per_w
        NQ = 4
        sizes = (96, 160, 160, 96)
        offs = (0, 96, 256, 416)
        bufs = (b0, b1, b2, b3)
        sems = (s0, s1, s2, s3)
        c_sty = pltpu.async_copy(style_hbm.at[pl.ds(base, b_per_w)], sty_v, s0)
        c_frm = pltpu.async_copy(frame_hbm.at[pl.ds(base, b_per_w)], idx_v, s1)

        n_stage = S // 5
        @pl.when(sid < 5)
        def _stage():
            pltpu.async_copy(mu_hbm.at[pl.ds(sid * n_stage, n_stage)],
                             mu_sh.at[pl.ds(sid * n_stage, n_stage)], s_st)

        c_sty.wait()
        c_frm.wait()

        def flats(q):
            def flat_body(i, _):
                sl = pl.ds(offs[q] + i * L, L)
                idx_v[sl] = sty_v[sl] * F + idx_v[sl]
                return 0
            lax.fori_loop(0, sizes[q] // L, flat_body, 0)

        def gather(q):
            return pltpu.async_copy(
                table_hbm.at[idx_v.at[pl.ds(offs[q], sizes[q])]], bufs[q],
                sems[q])

        def mu_add(q):
            return pltpu.async_copy(
                mu_sh.at[sty_v.at[pl.ds(offs[q], sizes[q])]], bufs[q],
                sems[q], add=True)

        def write(q):
            return pltpu.async_copy(
                bufs[q], out_hbm.at[pl.ds(base + offs[q], sizes[q])], sems[q])

        flats(0)
        gs = {0: gather(0)}
        flats(1)
        gs[1] = gather(1)
        for q in range(2, NQ):
            flats(q)

        @pl.when(sid < 5)
        def _stage_drain():
            pltpu.make_async_copy(
                mu_hbm.at[pl.ds(sid * n_stage, n_stage)],
                mu_sh.at[pl.ds(sid * n_stage, n_stage)], s_st).wait()

        plsc.subcore_barrier()
        adds = {}
        ws = {}
        for q in range(NQ):
            gs[q].wait()
            adds[q] = mu_add(q)
            if q + 2 < NQ:
                gs[q + 2] = gather(q + 2)
            adds[q].wait()
            ws[q] = write(q)
        for q in range(NQ):
            ws[q].wait()

    return run(style_ids, frame_ids, flat_table, latents_mu)


# speedup vs baseline: 1.0041x; 1.0041x over previous
import functools

import jax
import jax.numpy as jnp
from jax import lax
from jax.experimental import pallas as pl
from jax.experimental.pallas import tpu as pltpu
from jax.experimental.pallas import tpu_sc as plsc


def kernel(style_ids, frame_ids, latents, latents_mu):
    S, F, D = latents.shape
    B = style_ids.shape[0]
    flat_table = latents.reshape(S * F, D)
    style_ids = style_ids.astype(jnp.int32)
    frame_ids = frame_ids.astype(jnp.int32)

    info = plsc.get_sparse_core_info()
    NC, NS, L = info.num_cores, info.num_subcores, info.num_lanes
    NW = NC * NS
    b_per_w = B // NW

    mesh = plsc.VectorSubcoreMesh(core_axis_name="c", subcore_axis_name="s")

    @functools.partial(
        pl.kernel,
        mesh=mesh,
        out_type=jax.ShapeDtypeStruct((B, D), jnp.float32),
        scratch_types=[
            pltpu.VMEM((b_per_w,), jnp.int32),
            pltpu.VMEM((b_per_w,), jnp.int32),
            pltpu.VMEM((160, D), jnp.float32),
            pltpu.VMEM((160, D), jnp.float32),
            pltpu.VMEM((128, D), jnp.float32),
            pltpu.VMEM((64, D), jnp.float32),
            pltpu.VMEM_SHARED((S, D), jnp.float32),
            pltpu.SemaphoreType.DMA,
            pltpu.SemaphoreType.DMA,
            pltpu.SemaphoreType.DMA,
            pltpu.SemaphoreType.DMA,
            pltpu.SemaphoreType.DMA,
        ],
    )
    def run(style_hbm, frame_hbm, table_hbm, mu_hbm, out_hbm,
            sty_v, idx_v, b0, b1, b2, b3, mu_sh, s0, s1, s2, s3, s_st):
        sid = lax.axis_index("s")
        wid = sid * NC + lax.axis_index("c")
        base = wid * b_per_w
        NQ = 4
        sizes = (160, 160, 128, 64)
        offs = (0, 160, 320, 448)
        bufs = (b0, b1, b2, b3)
        sems = (s0, s1, s2, s3)
        c_sty = pltpu.async_copy(style_hbm.at[pl.ds(base, b_per_w)], sty_v, s0)
        c_frm = pltpu.async_copy(frame_hbm.at[pl.ds(base, b_per_w)], idx_v, s1)

        n_stage = S // 5
        @pl.when(sid < 5)
        def _stage():
            pltpu.async_copy(mu_hbm.at[pl.ds(sid * n_stage, n_stage)],
                             mu_sh.at[pl.ds(sid * n_stage, n_stage)], s_st)

        c_sty.wait()
        c_frm.wait()

        def flats(q):
            def flat_body(i, _):
                sl = pl.ds(offs[q] + i * L, L)
                idx_v[sl] = sty_v[sl] * F + idx_v[sl]
                return 0
            lax.fori_loop(0, sizes[q] // L, flat_body, 0)

        def gather(q):
            return pltpu.async_copy(
                table_hbm.at[idx_v.at[pl.ds(offs[q], sizes[q])]], bufs[q],
                sems[q])

        def mu_add(q):
            return pltpu.async_copy(
                mu_sh.at[sty_v.at[pl.ds(offs[q], sizes[q])]], bufs[q],
                sems[q], add=True)

        def write(q):
            return pltpu.async_copy(
                bufs[q], out_hbm.at[pl.ds(base + offs[q], sizes[q])], sems[q])

        flats(0)
        gs = {0: gather(0)}
        flats(1)
        gs[1] = gather(1)
        for q in range(2, NQ):
            flats(q)

        @pl.when(sid < 5)
        def _stage_drain():
            pltpu.make_async_copy(
                mu_hbm.at[pl.ds(sid * n_stage, n_stage)],
                mu_sh.at[pl.ds(sid * n_stage, n_stage)], s_st).wait()

        plsc.subcore_barrier()
        adds = {}
        ws = {}
        for q in range(NQ):
            gs[q].wait()
            adds[q] = mu_add(q)
            if q + 2 < NQ:
                gs[q + 2] = gather(q + 2)
            adds[q].wait()
            ws[q] = write(q)
        for q in range(NQ):
            ws[q].wait()

    return run(style_ids, frame_ids, flat_table, latents_mu)
